# baseline (speedup 1.0000x reference)
import functools

import jax
import jax.numpy as jnp
from jax import lax
from jax.experimental import pallas as pl
from jax.experimental.pallas import tpu as pltpu
from jax.experimental.pallas import tpu_sc as plsc

_GAMMA = 24.0
_NC = 2
_NS = 16
_NW = _NC * _NS
_L = 16
_B = 4096
_NNEG = 128
_D = 64
_BW = _B // _NW
_NBUF = 4


def _gqe_body(ent, rel, pos_i, neg_i, qe_i, qr_i,
              pos_out, neg_out,
              qe_v, qr_v, pos_idx_v, neg_idx_v,
              q_rows, r_rows, pos_rows,
              bufs, out_pos_v, out_neg_v,
              sem_q, sem_r, sem_p, neg_sems):
    wid = lax.axis_index("s") * _NC + lax.axis_index("c")
    base = wid * _BW

    pltpu.sync_copy(qe_i.at[pl.ds(base, _BW)], qe_v)
    pltpu.sync_copy(qr_i.at[pl.ds(base, _BW)], qr_v)
    pltpu.sync_copy(pos_i.at[pl.ds(base, _BW)], pos_idx_v)
    pltpu.sync_copy(neg_i.at[pl.ds(base, _BW)], neg_idx_v)

    cq = pltpu.make_async_copy(ent.at[qe_v], q_rows, sem_q)
    cr = pltpu.make_async_copy(rel.at[qr_v], r_rows, sem_r)
    cp = pltpu.make_async_copy(ent.at[pos_idx_v], pos_rows, sem_p)
    cq.start()
    cr.start()
    cp.start()

    def neg_copy(b, k):
        return pltpu.make_async_copy(ent.at[neg_idx_v.at[b]], bufs[k],
                                     neg_sems[k])

    for k in range(_NBUF):
        neg_copy(k, k).start()

    cq.wait()
    cr.wait()
    cp.wait()

    def add_rel(b, _):
        for s in range(_D // _L):
            sl = pl.ds(s * _L, _L)
            q_rows[b, sl] = q_rows[b, sl] + r_rows[b, sl]
        return 0
    lax.fori_loop(0, _BW, add_rel, 0)

    for g in range(_BW // _L):
        b_ids = g * _L + lax.iota(jnp.int32, _L)

        def pbody(d, acc):
            dd = jnp.full((_L,), d, jnp.int32)
            vals = plsc.load_gather(pos_rows, [b_ids, dd])
            qv = plsc.load_gather(q_rows, [b_ids, dd])
            return acc + jnp.abs(vals - qv)

        acc = lax.fori_loop(0, _D, pbody, jnp.zeros((_L,), jnp.float32),
                            unroll=8)
        out_pos_v[pl.ds(g * _L, _L)] = _GAMMA - acc

    n_grp = _NNEG // _L
    grp_rows = [g * _L + lax.iota(jnp.int32, _L) for g in range(n_grp)]

    def compute_batch(b, buf):
        bb = jnp.full((_L,), b, jnp.int32)

        def nbody(d, accs):
            dd = jnp.full((_L,), d, jnp.int32)
            qb = plsc.load_gather(q_rows, [bb, dd])
            out = []
            for g in range(n_grp):
                vals = plsc.load_gather(buf, [grp_rows[g], dd])
                out.append(accs[g] + jnp.abs(vals - qb))
            return tuple(out)

        accs = lax.fori_loop(
            0, _D, nbody,
            tuple(jnp.zeros((_L,), jnp.float32) for _ in range(n_grp)),
            unroll=4)
        for g in range(n_grp):
            out_neg_v[b, pl.ds(g * _L, _L)] = _GAMMA - accs[g]

    def outer(it, _):
        for k in range(_NBUF):
            b = it * _NBUF + k
            neg_copy(b, k).wait()
            compute_batch(b, bufs[k])

            @pl.when(b + _NBUF < _BW)
            def _():
                neg_copy(b + _NBUF, k).start()
        return 0

    lax.fori_loop(0, _BW // _NBUF, outer, 0)

    pltpu.sync_copy(out_pos_v, pos_out.at[pl.ds(base, _BW)])
    pltpu.sync_copy(out_neg_v, neg_out.at[pl.ds(base, _BW)])


@functools.cache
def _build():
    mesh = plsc.VectorSubcoreMesh(core_axis_name="c", subcore_axis_name="s")
    scratch = [
        pltpu.VMEM((_BW,), jnp.int32),
        pltpu.VMEM((_BW,), jnp.int32),
        pltpu.VMEM((_BW,), jnp.int32),
        pltpu.VMEM((_BW, _NNEG), jnp.int32),
        pltpu.VMEM((_BW, _D), jnp.float32),
        pltpu.VMEM((_BW, _D), jnp.float32),
        pltpu.VMEM((_BW, _D), jnp.float32),
        [pltpu.VMEM((_NNEG, _D), jnp.float32) for _ in range(_NBUF)],
        pltpu.VMEM((_BW,), jnp.float32),
        pltpu.VMEM((_BW, _NNEG), jnp.float32),
        pltpu.SemaphoreType.DMA,
        pltpu.SemaphoreType.DMA,
        pltpu.SemaphoreType.DMA,
        [pltpu.SemaphoreType.DMA for _ in range(_NBUF)],
    ]
    return pl.kernel(
        _gqe_body,
        out_type=(
            jax.ShapeDtypeStruct((_B,), jnp.float32),
            jax.ShapeDtypeStruct((_B, _NNEG), jnp.float32),
        ),
        mesh=mesh,
        scratch_types=scratch,
        compiler_params=pltpu.CompilerParams(needs_layout_passes=False,
                                             use_tc_tiling_on_sc=False),
    )


def kernel(entity_table, relation_table, positive_sample, negative_sample,
           q_entity, q_relation):
    return _build()(entity_table, relation_table, positive_sample,
                    negative_sample, q_entity, q_relation)


# speedup vs baseline: 1.7040x; 1.7040x over previous
import functools

import jax
import jax.numpy as jnp
from jax import lax
from jax.experimental import pallas as pl
from jax.experimental.pallas import tpu as pltpu
from jax.experimental.pallas import tpu_sc as plsc

_GAMMA = 24.0
_NC = 2
_NS = 16
_NW = _NC * _NS
_L = 16
_B = 4096
_NNEG = 128
_D = 64
_BW = _B // _NW
_NBUF = 4


def _gqe_body(ent, rel, pos_i, neg_i, qe_i, qr_i,
              pos_out, neg_out,
              qe_v, qr_v, pos_idx_v, neg_idx_v,
              q_rows, r_rows, pos_rows,
              bufs, out_pos_v, out_neg_v,
              sem_q, sem_r, sem_p, neg_sems):
    wid = lax.axis_index("s") * _NC + lax.axis_index("c")
    base = wid * _BW

    pltpu.sync_copy(qe_i.at[pl.ds(base, _BW)], qe_v)
    pltpu.sync_copy(qr_i.at[pl.ds(base, _BW)], qr_v)
    pltpu.sync_copy(pos_i.at[pl.ds(base, _BW)], pos_idx_v)
    pltpu.sync_copy(neg_i.at[pl.ds(base, _BW)], neg_idx_v)

    cq = pltpu.make_async_copy(ent.at[qe_v], q_rows, sem_q)
    cr = pltpu.make_async_copy(rel.at[qr_v], r_rows, sem_r)
    cp = pltpu.make_async_copy(ent.at[pos_idx_v], pos_rows, sem_p)
    cq.start()
    cr.start()
    cp.start()

    def neg_copy(b, k):
        return pltpu.make_async_copy(ent.at[neg_idx_v.at[b]], bufs[k],
                                     neg_sems[k])

    for k in range(_NBUF):
        neg_copy(k, k).start()

    cq.wait()
    cr.wait()
    cp.wait()

    def add_rel(b, _):
        for s in range(_D // _L):
            sl = pl.ds(s * _L, _L)
            q_rows[b, sl] = q_rows[b, sl] + r_rows[b, sl]
        return 0
    lax.fori_loop(0, _BW, add_rel, 0)

    iota = lax.iota(jnp.int32, _L)


    for g in range(_BW // _L):
        b_ids = g * _L + iota

        def pbody(d, acc):
            col = jnp.bitwise_and(d + iota, _D - 1)
            vals = plsc.load_gather(pos_rows, [b_ids, col])
            qv = plsc.load_gather(q_rows, [b_ids, col])
            return acc + jnp.abs(vals - qv)

        acc = lax.fori_loop(0, _D, pbody, jnp.zeros((_L,), jnp.float32),
                            unroll=4)
        out_pos_v[pl.ds(g * _L, _L)] = _GAMMA - acc

    n_grp = _NNEG // _L
    grp_rows = [g * _L + iota for g in range(n_grp)]

    def compute_batch(b, buf):
        bb = jnp.full((_L,), b, jnp.int32)

        def nbody(d, accs):
            col = jnp.bitwise_and(d + iota, _D - 1)
            qb = plsc.load_gather(q_rows, [bb, col])
            out = []
            for g in range(n_grp):
                vals = plsc.load_gather(buf, [grp_rows[g], col])
                out.append(accs[g] + jnp.abs(vals - qb))
            return tuple(out)

        accs = lax.fori_loop(
            0, _D, nbody,
            tuple(jnp.zeros((_L,), jnp.float32) for _ in range(n_grp)),
            unroll=2)
        for g in range(n_grp):
            out_neg_v[b, pl.ds(g * _L, _L)] = _GAMMA - accs[g]

    def outer(it, _):
        for k in range(_NBUF):
            b = it * _NBUF + k
            neg_copy(b, k).wait()
            compute_batch(b, bufs[k])

            @pl.when(b + _NBUF < _BW)
            def _():
                neg_copy(b + _NBUF, k).start()
        return 0

    lax.fori_loop(0, _BW // _NBUF, outer, 0)

    pltpu.sync_copy(out_pos_v, pos_out.at[pl.ds(base, _BW)])
    pltpu.sync_copy(out_neg_v, neg_out.at[pl.ds(base, _BW)])


@functools.cache
def _build():
    mesh = plsc.VectorSubcoreMesh(core_axis_name="c", subcore_axis_name="s")
    scratch = [
        pltpu.VMEM((_BW,), jnp.int32),
        pltpu.VMEM((_BW,), jnp.int32),
        pltpu.VMEM((_BW,), jnp.int32),
        pltpu.VMEM((_BW, _NNEG), jnp.int32),
        pltpu.VMEM((_BW, _D), jnp.float32),
        pltpu.VMEM((_BW, _D), jnp.float32),
        pltpu.VMEM((_BW, _D), jnp.float32),
        [pltpu.VMEM((_NNEG, _D), jnp.float32) for _ in range(_NBUF)],
        pltpu.VMEM((_BW,), jnp.float32),
        pltpu.VMEM((_BW, _NNEG), jnp.float32),
        pltpu.SemaphoreType.DMA,
        pltpu.SemaphoreType.DMA,
        pltpu.SemaphoreType.DMA,
        [pltpu.SemaphoreType.DMA for _ in range(_NBUF)],
    ]
    return pl.kernel(
        _gqe_body,
        out_type=(
            jax.ShapeDtypeStruct((_B,), jnp.float32),
            jax.ShapeDtypeStruct((_B, _NNEG), jnp.float32),
        ),
        mesh=mesh,
        scratch_types=scratch,
        compiler_params=pltpu.CompilerParams(needs_layout_passes=False,
                                             use_tc_tiling_on_sc=False),
    )


def kernel(entity_table, relation_table, positive_sample, negative_sample,
           q_entity, q_relation):
    return _build()(entity_table, relation_table, positive_sample,
                    negative_sample, q_entity, q_relation)
